# baseline (speedup 1.0000x reference)
import functools

import jax
import jax.numpy as jnp
from jax.experimental import pallas as pl
from jax.experimental.pallas import tpu as pltpu

ROW_BLOCK = 256


def _fused_kernel(
    a_ref, o_ref, heavy_ref, v_ref, *, n_row_blocks, k, recent, min_value
):
    r = pl.program_id(1)
    n = a_ref.shape[2]

    @pl.when(r == 0)
    def _():
        def colsum_body(c, acc):
            a = a_ref[0, pl.ds(c * ROW_BLOCK, ROW_BLOCK), :]
            m = jnp.max(a, axis=1, keepdims=True)
            e = jnp.exp(a - m)
            s = jnp.sum(e, axis=1, keepdims=True)
            return acc + jnp.sum(e / s, axis=0, keepdims=True)

        v = jax.lax.fori_loop(
            0, n_row_blocks, colsum_body, jnp.zeros((1, n), jnp.float32)
        )
        v_ref[...] = v

        def rank_body(c, acc):
            vc = v_ref[:, pl.ds(c * ROW_BLOCK, ROW_BLOCK)].reshape(ROW_BLOCK, 1)
            ii = (
                jax.lax.broadcasted_iota(jnp.int32, (ROW_BLOCK, n), 0)
                + c * ROW_BLOCK
            )
            jj = jax.lax.broadcasted_iota(jnp.int32, (ROW_BLOCK, n), 1)
            beats = (vc > v) | ((vc == v) & (ii < jj))
            return acc + jnp.sum(beats.astype(jnp.int32), axis=0, keepdims=True)

        rank = jax.lax.fori_loop(
            0, n_row_blocks, rank_body, jnp.zeros((1, n), jnp.int32)
        )
        heavy_ref[...] = (rank < k).astype(jnp.int32)

    a = a_ref[0, pl.ds(r * ROW_BLOCK, ROW_BLOCK), :]
    hv = heavy_ref[...]
    i = jax.lax.broadcasted_iota(jnp.int32, (ROW_BLOCK, n), 0) + r * ROW_BLOCK
    j = jax.lax.broadcasted_iota(jnp.int32, (ROW_BLOCK, n), 1)
    band = (j <= i + recent) & (j >= i - recent)
    keep = band | (hv != 0)
    o_ref[0] = jnp.where(keep, a, jnp.float32(min_value))


def kernel(attn_weights):
    bs, head, query_len, key_len = attn_weights.shape
    heavy_budget = min(int(0.1 * key_len), key_len)
    recent_budget = int(0.1 * key_len)
    min_value = float(jnp.finfo(attn_weights.dtype).min)

    a = attn_weights.reshape(bs * head, query_len, key_len)
    nh = bs * head
    n_row_blocks = query_len // ROW_BLOCK

    out = pl.pallas_call(
        functools.partial(
            _fused_kernel,
            n_row_blocks=n_row_blocks,
            k=heavy_budget,
            recent=recent_budget,
            min_value=min_value,
        ),
        grid=(nh, n_row_blocks),
        in_specs=[
            pl.BlockSpec((1, query_len, key_len), lambda h, r: (h, 0, 0)),
        ],
        out_specs=pl.BlockSpec((1, ROW_BLOCK, key_len), lambda h, r: (h, r, 0)),
        out_shape=jax.ShapeDtypeStruct((nh, query_len, key_len), jnp.float32),
        scratch_shapes=[
            pltpu.VMEM((1, key_len), jnp.int32),
            pltpu.VMEM((1, key_len), jnp.float32),
        ],
    )(a)

    return out.reshape(bs, head, query_len, key_len)


# speedup vs baseline: 1.0243x; 1.0243x over previous
import functools

import jax
import jax.numpy as jnp
from jax.experimental import pallas as pl
from jax.experimental.pallas import tpu as pltpu

ROW_BLOCK = 256


def _fused_kernel(
    a_ref, o_ref, heavy_ref, v_ref, *, n_row_blocks, k, recent, min_value
):
    r = pl.program_id(1)
    n = a_ref.shape[2]

    @pl.when(r == 0)
    def _():
        def colsum_body(c, acc):
            a = a_ref[0, pl.ds(c * ROW_BLOCK, ROW_BLOCK), :]
            m = jnp.max(a, axis=1, keepdims=True)
            e = jnp.exp(a - m)
            s = jnp.sum(e, axis=1, keepdims=True)
            return acc + jnp.sum(e / s, axis=0, keepdims=True)

        v = jax.lax.fori_loop(
            0, n_row_blocks, colsum_body, jnp.zeros((1, n), jnp.float32)
        )
        v_ref[...] = v

        vb = jax.lax.bitcast_convert_type(v, jnp.int32)

        def tsearch_body(b, t):
            cand = t | jax.lax.shift_left(jnp.int32(1), jnp.int32(30) - b)
            cnt = jnp.sum((vb >= cand).astype(jnp.int32))
            return jnp.where(cnt >= k, cand, t)

        t = jax.lax.fori_loop(0, 31, tsearch_body, jnp.int32(0))

        gt = vb > t
        eq = vb == t
        m = k - jnp.sum(gt.astype(jnp.int32))
        idx = jax.lax.broadcasted_iota(jnp.int32, (1, n), 1)
        w = jnp.where(eq, idx, jnp.int32(2 * n))

        def isearch_body(b, x):
            cand = x & ~jax.lax.shift_left(jnp.int32(1), jnp.int32(11) - b)
            cnt = jnp.sum((w <= cand).astype(jnp.int32))
            return jnp.where(cnt >= m, cand, x)

        x = jax.lax.fori_loop(0, 12, isearch_body, jnp.int32(4095))

        heavy_ref[...] = (gt | (eq & (idx <= x))).astype(jnp.int32)

    a = a_ref[0, pl.ds(r * ROW_BLOCK, ROW_BLOCK), :]
    hv = heavy_ref[...]
    i = jax.lax.broadcasted_iota(jnp.int32, (ROW_BLOCK, n), 0) + r * ROW_BLOCK
    j = jax.lax.broadcasted_iota(jnp.int32, (ROW_BLOCK, n), 1)
    band = (j <= i + recent) & (j >= i - recent)
    keep = band | (hv != 0)
    o_ref[0] = jnp.where(keep, a, jnp.float32(min_value))


def kernel(attn_weights):
    bs, head, query_len, key_len = attn_weights.shape
    heavy_budget = min(int(0.1 * key_len), key_len)
    recent_budget = int(0.1 * key_len)
    min_value = float(jnp.finfo(attn_weights.dtype).min)

    a = attn_weights.reshape(bs * head, query_len, key_len)
    nh = bs * head
    n_row_blocks = query_len // ROW_BLOCK

    out = pl.pallas_call(
        functools.partial(
            _fused_kernel,
            n_row_blocks=n_row_blocks,
            k=heavy_budget,
            recent=recent_budget,
            min_value=min_value,
        ),
        grid=(nh, n_row_blocks),
        in_specs=[
            pl.BlockSpec((1, query_len, key_len), lambda h, r: (h, 0, 0)),
        ],
        out_specs=pl.BlockSpec((1, ROW_BLOCK, key_len), lambda h, r: (h, r, 0)),
        out_shape=jax.ShapeDtypeStruct((nh, query_len, key_len), jnp.float32),
        scratch_shapes=[
            pltpu.VMEM((1, key_len), jnp.int32),
            pltpu.VMEM((1, key_len), jnp.float32),
        ],
    )(a)

    return out.reshape(bs, head, query_len, key_len)
